# baseline (speedup 1.0000x reference)
import functools

import jax
import jax.numpy as jnp
from jax import lax
from jax.experimental import pallas as pl
from jax.experimental.pallas import tpu as pltpu
from jax.experimental.pallas import tpu_sc as plsc

N = 10000
E = 320000
H = 128
NH = 64
A = 256
R = 10240
NW = 32
CH = 128
PH = 4
KP0 = 24
KP1 = 16
B1 = 16 * PH * KP0
ROWS_E = 16 * PH * (KP0 + KP1)
EPAD = ROWS_E * CH
BR = 2048
STRIPE = R // 16

_MESH = dict(core_axis_name="c", subcore_axis_name="s", num_cores=2,
             num_subcores=16)


def _sc_aggregate(x, src, dst, z2, z1):
    mesh = plsc.VectorSubcoreMesh(**_MESH)

    @functools.partial(
        pl.kernel,
        out_type=(jax.ShapeDtypeStruct((2, R, H), jnp.float32),
                  jax.ShapeDtypeStruct((2, R), jnp.float32)),
        mesh=mesh,
        scratch_types=[
            pltpu.VMEM((KP0, CH), jnp.int32),
            pltpu.VMEM((KP0, CH), jnp.int32),
            pltpu.VMEM((2, CH, H), jnp.float32),
            pltpu.VMEM((CH,), jnp.float32),
            pltpu.VMEM_SHARED((R, H), jnp.float32),
            pltpu.VMEM_SHARED((R,), jnp.float32),
            pltpu.SemaphoreType.DMA,
            pltpu.SemaphoreType.DMA,
            pltpu.SemaphoreType.DMA,
        ],
    )
    def kern(x_hbm, src_hbm, dst_hbm, z2_hbm, z1_hbm, acc_out, deg_out,
             src_v, dst_v, rows_v, ones_v, acc_sh, deg_sh, gsem, ssem, dsem):
        cid = lax.axis_index("c")
        sid = lax.axis_index("s")
        pltpu.sync_copy(z2_hbm.at[pl.ds(sid * STRIPE, STRIPE)],
                        acc_sh.at[pl.ds(sid * STRIPE, STRIPE)])
        pltpu.sync_copy(z1_hbm.at[pl.ds(sid * STRIPE, STRIPE)],
                        deg_sh.at[pl.ds(sid * STRIPE, STRIPE)])
        for j in range(CH // 16):
            ones_v[pl.ds(j * 16, 16)] = jnp.ones((16,), jnp.float32)
        plsc.subcore_barrier()

        def gather(m):
            return pltpu.make_async_copy(
                x_hbm.at[src_v.at[m]], rows_v.at[m % 2], gsem)

        def acc_scatter(m):
            return pltpu.make_async_copy(
                rows_v.at[m % 2], acc_sh.at[dst_v.at[m]], ssem)

        def deg_scatter(m):
            return pltpu.make_async_copy(
                ones_v, deg_sh.at[dst_v.at[m]], dsem)

        def run(base, kp):
            for phase in range(PH):
                row0 = base + phase * kp
                pltpu.sync_copy(src_hbm.at[pl.ds(row0, kp)],
                                src_v.at[pl.ds(0, kp)])
                pltpu.sync_copy(dst_hbm.at[pl.ds(row0, kp)],
                                dst_v.at[pl.ds(0, kp)])
                gather(0).start()

                def body(m, carry):
                    gather(m + 1).start()
                    gather(m).wait()
                    deg_scatter(m).start(add=True)
                    acc_scatter(m).start(add=True)
                    acc_scatter(m).wait()
                    return carry

                lax.fori_loop(0, kp - 1, body, 0)
                gather(kp - 1).wait()
                deg_scatter(kp - 1).start(add=True)
                acc_scatter(kp - 1).start(add=True)
                acc_scatter(kp - 1).wait()

                def drain(m, carry):
                    deg_scatter(0).wait()
                    return carry

                lax.fori_loop(0, kp, drain, 0)

        @pl.when(cid == 0)
        def _():
            run(sid * (PH * KP0), KP0)

        @pl.when(cid == 1)
        def _():
            run(B1 + sid * (PH * KP1), KP1)

        plsc.subcore_barrier()
        pltpu.sync_copy(acc_sh.at[pl.ds(sid * STRIPE, STRIPE)],
                        acc_out.at[cid, pl.ds(sid * STRIPE, STRIPE)])
        pltpu.sync_copy(deg_sh.at[pl.ds(sid * STRIPE, STRIPE)],
                        deg_out.at[cid, pl.ds(sid * STRIPE, STRIPE)])

    return kern(x, src, dst, z2, z1)


def _conv_math(acc_ref, deg_ref, x_ref, wl_ref, wr_ref, b_ref):
    agg = acc_ref[0] + acc_ref[1]
    deg = deg_ref[0] + deg_ref[1]
    mean = agg / jnp.maximum(deg, 1.0)
    return (jnp.dot(mean, wl_ref[...], preferred_element_type=jnp.float32)
            + jnp.dot(x_ref[...], wr_ref[...],
                      preferred_element_type=jnp.float32)
            + b_ref[...])


def _tc_conv(accp, degp, x, wl, wr, b):
    def body(acc_ref, deg_ref, x_ref, wl_ref, wr_ref, b_ref, out_ref):
        out_ref[...] = _conv_math(acc_ref, deg_ref, x_ref, wl_ref, wr_ref,
                                  b_ref)

    w_spec = pl.BlockSpec((H, H), lambda i: (0, 0))
    return pl.pallas_call(
        body,
        grid=(R // BR,),
        in_specs=[
            pl.BlockSpec((2, BR, H), lambda i: (0, i, 0)),
            pl.BlockSpec((2, BR, 1), lambda i: (0, i, 0)),
            pl.BlockSpec((BR, H), lambda i: (i, 0)),
            w_spec, w_spec,
            pl.BlockSpec((1, H), lambda i: (0, 0)),
        ],
        out_specs=pl.BlockSpec((BR, H), lambda i: (i, 0)),
        out_shape=jax.ShapeDtypeStruct((R, H), jnp.float32),
    )(accp, degp.reshape(2, R, 1), x, wl, wr, b.reshape(1, H))


def _tc_final(accp, degp, x, wl, wr, b, w0, c0, w1, c1, w2, c2):
    def body(acc_ref, deg_ref, x_ref, wl_ref, wr_ref, b_ref,
             w0_ref, c0_ref, w1_ref, c1_ref, w2_ref, c2_ref, out_ref):
        y = _conv_math(acc_ref, deg_ref, x_ref, wl_ref, wr_ref, b_ref)
        y = jnp.dot(y, w0_ref[...], preferred_element_type=jnp.float32) + c0_ref[...]
        y = jnp.dot(y, w1_ref[...], preferred_element_type=jnp.float32) + c1_ref[...]
        y = jnp.dot(y, w2_ref[...], preferred_element_type=jnp.float32) + c2_ref[...]
        out_ref[...] = y

    w_spec = pl.BlockSpec((H, H), lambda i: (0, 0))
    b_spec = pl.BlockSpec((1, H), lambda i: (0, 0))
    return pl.pallas_call(
        body,
        grid=(R // BR,),
        in_specs=[
            pl.BlockSpec((2, BR, H), lambda i: (0, i, 0)),
            pl.BlockSpec((2, BR, 1), lambda i: (0, i, 0)),
            pl.BlockSpec((BR, H), lambda i: (i, 0)),
            w_spec, w_spec, b_spec,
            w_spec, b_spec, w_spec, b_spec, w_spec, b_spec,
        ],
        out_specs=pl.BlockSpec((BR, H), lambda i: (i, 0)),
        out_shape=jax.ShapeDtypeStruct((R, H), jnp.float32),
    )(accp, degp.reshape(2, R, 1), x, wl, wr, b.reshape(1, H), w0, c0.reshape(1, H),
      w1, c1.reshape(1, H), w2, c2.reshape(1, H))


def _sc_gather(y, idx):
    bpw = (2 * A) // NW
    mesh = plsc.VectorSubcoreMesh(**_MESH)

    @functools.partial(
        pl.kernel,
        out_type=jax.ShapeDtypeStruct((2 * A, H), jnp.float32),
        mesh=mesh,
        scratch_types=[
            pltpu.VMEM((1, bpw), jnp.int32),
            pltpu.VMEM((bpw, H), jnp.float32),
        ],
    )
    def kern(y_hbm, idx_hbm, out_hbm, idx_v, rows_v):
        cid = lax.axis_index("c")
        sid = lax.axis_index("s")
        base = (sid * 2 + cid) * bpw
        pltpu.sync_copy(idx_hbm.at[pl.ds(base, bpw)], idx_v.at[0])
        pltpu.sync_copy(y_hbm.at[idx_v.at[0]], rows_v)
        pltpu.sync_copy(rows_v, out_hbm.at[pl.ds(base, bpw)])

    return kern(y, idx)


def _tc_actions(g):
    def body(g_ref, out_ref):
        prod = g_ref[0:A, 0:NH] * g_ref[A:2 * A, NH:H]
        p = jnp.sum(prod, axis=0)
        m = jnp.max(p)
        e = jnp.exp(p - m)
        out_ref[0, :] = e / jnp.sum(e)

    return pl.pallas_call(
        body,
        out_shape=jax.ShapeDtypeStruct((1, NH), jnp.float32),
    )(g)


def kernel(actions, obs, eic, eid, eit, batch,
           W_l1, W_r1, b1, W_l2, W_r2, b2, W_l3, W_r3, b3,
           W_lin0, b_lin0, W_lin1, b_lin1, W_out, b_out):
    x = jnp.pad(obs, ((0, R - N), (0, 0)))
    z2 = jnp.zeros((R, H), jnp.float32)
    z1 = jnp.zeros((R,), jnp.float32)
    npad = EPAD - E
    gsrc = jnp.arange(npad, dtype=jnp.int32) % N
    gdst = N + (jnp.arange(npad, dtype=jnp.int32) % (R - N))

    def prep(e):
        s = jnp.concatenate([e[0], gsrc])
        d = jnp.concatenate([e[1], gdst])
        return s.reshape(ROWS_E, CH), d.reshape(ROWS_E, CH)

    s1, d1 = prep(eit)
    s2, d2 = prep(eic)
    s3, d3 = prep(eid)

    accp, degp = _sc_aggregate(x, s1, d1, z2, z1)
    x1 = _tc_conv(accp, degp, x, W_l1, W_r1, b1)
    accp, degp = _sc_aggregate(x1, s2, d2, z2, z1)
    x2 = _tc_conv(accp, degp, x1, W_l2, W_r2, b2)
    accp, degp = _sc_aggregate(x2, s3, d3, z2, z1)
    y = _tc_final(accp, degp, x2, W_l3, W_r3, b3,
                  W_lin0, b_lin0, W_lin1, b_lin1, W_out, b_out)

    idx = jnp.concatenate([actions[0, :, 0], actions[0, :, 1]])
    g = _sc_gather(y, idx.astype(jnp.int32))
    return _tc_actions(g)


# speedup vs baseline: 1.0501x; 1.0501x over previous
import functools

import jax
import jax.numpy as jnp
from jax import lax
from jax.experimental import pallas as pl
from jax.experimental.pallas import tpu as pltpu
from jax.experimental.pallas import tpu_sc as plsc

N = 10000
E = 320000
H = 128
NH = 64
A = 256
R = 10240
NW = 32
CH = 128
PH = 4
KP0 = 24
KP1 = 16
B1 = 16 * PH * KP0
ROWS_E = 16 * PH * (KP0 + KP1)
EPAD = ROWS_E * CH
BR = 2048
STRIPE = R // 16

_MESH = dict(core_axis_name="c", subcore_axis_name="s", num_cores=2,
             num_subcores=16)


def _sc_aggregate(x, src, dst, z2, z1):
    mesh = plsc.VectorSubcoreMesh(**_MESH)

    @functools.partial(
        pl.kernel,
        out_type=(jax.ShapeDtypeStruct((2, R, H), jnp.float32),
                  jax.ShapeDtypeStruct((2, R), jnp.float32)),
        mesh=mesh,
        scratch_types=[
            pltpu.VMEM((KP0, CH), jnp.int32),
            pltpu.VMEM((KP0, CH), jnp.int32),
            pltpu.VMEM((2, CH, H), jnp.float32),
            pltpu.VMEM((CH,), jnp.float32),
            pltpu.VMEM_SHARED((R, H), jnp.float32),
            pltpu.VMEM_SHARED((R,), jnp.float32),
            pltpu.SemaphoreType.DMA,
            pltpu.SemaphoreType.DMA,
            pltpu.SemaphoreType.DMA,
        ],
    )
    def kern(x_hbm, src_hbm, dst_hbm, z2_hbm, z1_hbm, acc_out, deg_out,
             src_v, dst_v, rows_v, ones_v, acc_sh, deg_sh, gsem, ssem, dsem):
        cid = lax.axis_index("c")
        sid = lax.axis_index("s")
        pltpu.sync_copy(z2_hbm.at[pl.ds(sid * STRIPE, STRIPE)],
                        acc_sh.at[pl.ds(sid * STRIPE, STRIPE)])
        pltpu.sync_copy(z1_hbm.at[pl.ds(sid * STRIPE, STRIPE)],
                        deg_sh.at[pl.ds(sid * STRIPE, STRIPE)])
        for j in range(CH // 16):
            ones_v[pl.ds(j * 16, 16)] = jnp.ones((16,), jnp.float32)
        plsc.subcore_barrier()

        def gather(m):
            return pltpu.make_async_copy(
                x_hbm.at[src_v.at[m]], rows_v.at[m % 2], gsem)

        def acc_scatter(m):
            return pltpu.make_async_copy(
                rows_v.at[m % 2], acc_sh.at[dst_v.at[m]], ssem)

        def deg_scatter(m):
            return pltpu.make_async_copy(
                ones_v, deg_sh.at[dst_v.at[m]], dsem)

        def run(base, kp):
            for phase in range(PH):
                row0 = base + phase * kp
                pltpu.sync_copy(src_hbm.at[pl.ds(row0, kp)],
                                src_v.at[pl.ds(0, kp)])
                pltpu.sync_copy(dst_hbm.at[pl.ds(row0, kp)],
                                dst_v.at[pl.ds(0, kp)])
                gather(0).start()

                def body(m, carry):
                    gather(m + 1).start()
                    gather(m).wait()
                    deg_scatter(m).start(add=True)
                    acc_scatter(m).start(add=True)
                    acc_scatter(m).wait()
                    return carry

                lax.fori_loop(0, kp - 1, body, 0)
                gather(kp - 1).wait()
                deg_scatter(kp - 1).start(add=True)
                acc_scatter(kp - 1).start(add=True)
                acc_scatter(kp - 1).wait()

                def drain(m, carry):
                    deg_scatter(0).wait()
                    return carry

                lax.fori_loop(0, kp, drain, 0)

        @pl.when(cid == 0)
        def _():
            run(sid * (PH * KP0), KP0)

        @pl.when(cid == 1)
        def _():
            run(B1 + sid * (PH * KP1), KP1)

        plsc.subcore_barrier()
        pltpu.sync_copy(acc_sh.at[pl.ds(sid * STRIPE, STRIPE)],
                        acc_out.at[cid, pl.ds(sid * STRIPE, STRIPE)])
        pltpu.sync_copy(deg_sh.at[pl.ds(sid * STRIPE, STRIPE)],
                        deg_out.at[cid, pl.ds(sid * STRIPE, STRIPE)])

    return kern(x, src, dst, z2, z1)


def _conv_math(acc_ref, deg_ref, x_ref, wl_ref, wr_ref, b_ref):
    agg = acc_ref[0] + acc_ref[1]
    deg = deg_ref[0] + deg_ref[1]
    mean = agg / jnp.maximum(deg, 1.0)[:, None]
    return (jnp.dot(mean, wl_ref[...], preferred_element_type=jnp.float32)
            + jnp.dot(x_ref[...], wr_ref[...],
                      preferred_element_type=jnp.float32)
            + b_ref[...])


def _tc_conv(accp, degp, x, wl, wr, b):
    def body(acc_ref, deg_ref, x_ref, wl_ref, wr_ref, b_ref, out_ref):
        out_ref[...] = _conv_math(acc_ref, deg_ref, x_ref, wl_ref, wr_ref,
                                  b_ref)

    w_spec = pl.BlockSpec((H, H), lambda i: (0, 0))
    return pl.pallas_call(
        body,
        grid=(R // BR,),
        in_specs=[
            pl.BlockSpec((2, BR, H), lambda i: (0, i, 0)),
            pl.BlockSpec((2, BR), lambda i: (0, i)),
            pl.BlockSpec((BR, H), lambda i: (i, 0)),
            w_spec, w_spec,
            pl.BlockSpec((1, H), lambda i: (0, 0)),
        ],
        out_specs=pl.BlockSpec((BR, H), lambda i: (i, 0)),
        out_shape=jax.ShapeDtypeStruct((R, H), jnp.float32),
    )(accp, degp, x, wl, wr, b.reshape(1, H))


def _tc_final(accp, degp, x, wl, wr, b, w0, c0, w1, c1, w2, c2):
    def body(acc_ref, deg_ref, x_ref, wl_ref, wr_ref, b_ref,
             w0_ref, c0_ref, w1_ref, c1_ref, w2_ref, c2_ref, out_ref):
        y = _conv_math(acc_ref, deg_ref, x_ref, wl_ref, wr_ref, b_ref)
        y = jnp.dot(y, w0_ref[...], preferred_element_type=jnp.float32) + c0_ref[...]
        y = jnp.dot(y, w1_ref[...], preferred_element_type=jnp.float32) + c1_ref[...]
        y = jnp.dot(y, w2_ref[...], preferred_element_type=jnp.float32) + c2_ref[...]
        out_ref[...] = y

    w_spec = pl.BlockSpec((H, H), lambda i: (0, 0))
    b_spec = pl.BlockSpec((1, H), lambda i: (0, 0))
    return pl.pallas_call(
        body,
        grid=(R // BR,),
        in_specs=[
            pl.BlockSpec((2, BR, H), lambda i: (0, i, 0)),
            pl.BlockSpec((2, BR), lambda i: (0, i)),
            pl.BlockSpec((BR, H), lambda i: (i, 0)),
            w_spec, w_spec, b_spec,
            w_spec, b_spec, w_spec, b_spec, w_spec, b_spec,
        ],
        out_specs=pl.BlockSpec((BR, H), lambda i: (i, 0)),
        out_shape=jax.ShapeDtypeStruct((R, H), jnp.float32),
    )(accp, degp, x, wl, wr, b.reshape(1, H), w0, c0.reshape(1, H),
      w1, c1.reshape(1, H), w2, c2.reshape(1, H))


def _sc_gather(y, idx):
    bpw = (2 * A) // NW
    mesh = plsc.VectorSubcoreMesh(**_MESH)

    @functools.partial(
        pl.kernel,
        out_type=jax.ShapeDtypeStruct((2 * A, H), jnp.float32),
        mesh=mesh,
        scratch_types=[
            pltpu.VMEM((1, bpw), jnp.int32),
            pltpu.VMEM((bpw, H), jnp.float32),
        ],
    )
    def kern(y_hbm, idx_hbm, out_hbm, idx_v, rows_v):
        cid = lax.axis_index("c")
        sid = lax.axis_index("s")
        base = (sid * 2 + cid) * bpw
        pltpu.sync_copy(idx_hbm.at[pl.ds(base, bpw)], idx_v.at[0])
        pltpu.sync_copy(y_hbm.at[idx_v.at[0]], rows_v)
        pltpu.sync_copy(rows_v, out_hbm.at[pl.ds(base, bpw)])

    return kern(y, idx)


def _tc_actions(g):
    def body(g_ref, out_ref):
        prod = g_ref[0:A, 0:NH] * g_ref[A:2 * A, NH:H]
        p = jnp.sum(prod, axis=0)
        m = jnp.max(p)
        e = jnp.exp(p - m)
        out_ref[0, :] = e / jnp.sum(e)

    return pl.pallas_call(
        body,
        out_shape=jax.ShapeDtypeStruct((1, NH), jnp.float32),
    )(g)


def kernel(actions, obs, eic, eid, eit, batch,
           W_l1, W_r1, b1, W_l2, W_r2, b2, W_l3, W_r3, b3,
           W_lin0, b_lin0, W_lin1, b_lin1, W_out, b_out):
    x = jnp.pad(obs, ((0, R - N), (0, 0)))
    z2 = jnp.zeros((R, H), jnp.float32)
    z1 = jnp.zeros((R,), jnp.float32)
    npad = EPAD - E
    gsrc = jnp.arange(npad, dtype=jnp.int32) % N
    gdst = N + (jnp.arange(npad, dtype=jnp.int32) % (R - N))

    def prep(e):
        s = jnp.concatenate([e[0], gsrc])
        d = jnp.concatenate([e[1], gdst])
        return s.reshape(ROWS_E, CH), d.reshape(ROWS_E, CH)

    s1, d1 = prep(eit)
    s2, d2 = prep(eic)
    s3, d3 = prep(eid)

    accp, degp = _sc_aggregate(x, s1, d1, z2, z1)
    x1 = _tc_conv(accp, degp, x, W_l1, W_r1, b1)
    accp, degp = _sc_aggregate(x1, s2, d2, z2, z1)
    x2 = _tc_conv(accp, degp, x1, W_l2, W_r2, b2)
    accp, degp = _sc_aggregate(x2, s3, d3, z2, z1)
    y = _tc_final(accp, degp, x2, W_l3, W_r3, b3,
                  W_lin0, b_lin0, W_lin1, b_lin1, W_out, b_out)

    idx = jnp.concatenate([actions[0, :, 0], actions[0, :, 1]])
    g = _sc_gather(y, idx.astype(jnp.int32))
    return _tc_actions(g)


# speedup vs baseline: 1.1309x; 1.0769x over previous
import functools

import jax
import jax.numpy as jnp
from jax import lax
from jax.experimental import pallas as pl
from jax.experimental.pallas import tpu as pltpu
from jax.experimental.pallas import tpu_sc as plsc

N = 10000
E = 320000
H = 128
NH = 64
A = 256
R = 10240
NW = 32
CH = 128
PH = 4
KP0 = 21
KP1 = 19
LW = 32
B1 = 16 * PH * KP0
ROWS_E = 16 * PH * (KP0 + KP1) + 8
EPAD = ROWS_E * CH
BR = 2048
STRIPE = R // 16

_MESH = dict(core_axis_name="c", subcore_axis_name="s", num_cores=2,
             num_subcores=16)


def _sc_aggregate(x, src, dst, z2, z1):
    mesh = plsc.VectorSubcoreMesh(**_MESH)

    @functools.partial(
        pl.kernel,
        out_type=(jax.ShapeDtypeStruct((2, R, H), jnp.float32),
                  jax.ShapeDtypeStruct((2, R), jnp.float32)),
        mesh=mesh,
        scratch_types=[
            pltpu.VMEM((LW, CH), jnp.int32),
            pltpu.VMEM((LW, CH), jnp.int32),
            pltpu.VMEM((2, CH, H), jnp.float32),
            pltpu.VMEM((CH,), jnp.float32),
            pltpu.VMEM_SHARED((R, H), jnp.float32),
            pltpu.VMEM_SHARED((R,), jnp.float32),
            pltpu.SemaphoreType.DMA,
            pltpu.SemaphoreType.DMA,
            pltpu.SemaphoreType.DMA,
        ],
    )
    def kern(x_hbm, src_hbm, dst_hbm, z2_hbm, z1_hbm, acc_out, deg_out,
             src_v, dst_v, rows_v, ones_v, acc_sh, deg_sh, gsem, ssem, dsem):
        cid = lax.axis_index("c")
        sid = lax.axis_index("s")
        pltpu.sync_copy(z2_hbm.at[pl.ds(sid * STRIPE, STRIPE)],
                        acc_sh.at[pl.ds(sid * STRIPE, STRIPE)])
        pltpu.sync_copy(z1_hbm.at[pl.ds(sid * STRIPE, STRIPE)],
                        deg_sh.at[pl.ds(sid * STRIPE, STRIPE)])
        for j in range(CH // 16):
            ones_v[pl.ds(j * 16, 16)] = jnp.ones((16,), jnp.float32)
        plsc.subcore_barrier()

        def gather(m):
            return pltpu.make_async_copy(
                x_hbm.at[src_v.at[m]], rows_v.at[m % 2], gsem)

        def acc_scatter(m):
            return pltpu.make_async_copy(
                rows_v.at[m % 2], acc_sh.at[dst_v.at[m]], ssem)

        def deg_scatter(m):
            return pltpu.make_async_copy(
                ones_v, deg_sh.at[dst_v.at[m]], dsem)

        def run(base, kp):
            for phase in range(PH):
                row0 = base + phase * kp
                start = pl.multiple_of((row0 // 8) * 8, 8)
                off = row0 - start
                pltpu.sync_copy(src_hbm.at[pl.ds(start, LW)], src_v)
                pltpu.sync_copy(dst_hbm.at[pl.ds(start, LW)], dst_v)
                gather(off).start()

                def body(m, carry):
                    gather(m + 1).start()
                    gather(m).wait()
                    deg_scatter(m).start(add=True)
                    acc_scatter(m).start(add=True)
                    acc_scatter(m).wait()
                    return carry

                lax.fori_loop(off, off + kp - 1, body, 0)
                gather(off + kp - 1).wait()
                deg_scatter(off + kp - 1).start(add=True)
                acc_scatter(off + kp - 1).start(add=True)
                acc_scatter(off + kp - 1).wait()

                def drain(m, carry):
                    deg_scatter(0).wait()
                    return carry

                lax.fori_loop(0, kp, drain, 0)

        @pl.when(cid == 0)
        def _():
            run(sid * (PH * KP0), KP0)

        @pl.when(cid == 1)
        def _():
            run(B1 + sid * (PH * KP1), KP1)

        plsc.subcore_barrier()
        pltpu.sync_copy(acc_sh.at[pl.ds(sid * STRIPE, STRIPE)],
                        acc_out.at[cid, pl.ds(sid * STRIPE, STRIPE)])
        pltpu.sync_copy(deg_sh.at[pl.ds(sid * STRIPE, STRIPE)],
                        deg_out.at[cid, pl.ds(sid * STRIPE, STRIPE)])

    return kern(x, src, dst, z2, z1)


def _conv_math(acc_ref, deg_ref, x_ref, wl_ref, wr_ref, b_ref):
    agg = acc_ref[0] + acc_ref[1]
    deg = deg_ref[0] + deg_ref[1]
    mean = agg / jnp.maximum(deg, 1.0)[:, None]
    return (jnp.dot(mean, wl_ref[...], preferred_element_type=jnp.float32)
            + jnp.dot(x_ref[...], wr_ref[...],
                      preferred_element_type=jnp.float32)
            + b_ref[...])


def _tc_conv(accp, degp, x, wl, wr, b):
    def body(acc_ref, deg_ref, x_ref, wl_ref, wr_ref, b_ref, out_ref):
        out_ref[...] = _conv_math(acc_ref, deg_ref, x_ref, wl_ref, wr_ref,
                                  b_ref)

    w_spec = pl.BlockSpec((H, H), lambda i: (0, 0))
    return pl.pallas_call(
        body,
        grid=(R // BR,),
        in_specs=[
            pl.BlockSpec((2, BR, H), lambda i: (0, i, 0)),
            pl.BlockSpec((2, BR), lambda i: (0, i)),
            pl.BlockSpec((BR, H), lambda i: (i, 0)),
            w_spec, w_spec,
            pl.BlockSpec((1, H), lambda i: (0, 0)),
        ],
        out_specs=pl.BlockSpec((BR, H), lambda i: (i, 0)),
        out_shape=jax.ShapeDtypeStruct((R, H), jnp.float32),
    )(accp, degp, x, wl, wr, b.reshape(1, H))


def _tc_final(accp, degp, x, wl, wr, b, w0, c0, w1, c1, w2, c2):
    def body(acc_ref, deg_ref, x_ref, wl_ref, wr_ref, b_ref,
             w0_ref, c0_ref, w1_ref, c1_ref, w2_ref, c2_ref, out_ref):
        y = _conv_math(acc_ref, deg_ref, x_ref, wl_ref, wr_ref, b_ref)
        y = jnp.dot(y, w0_ref[...], preferred_element_type=jnp.float32) + c0_ref[...]
        y = jnp.dot(y, w1_ref[...], preferred_element_type=jnp.float32) + c1_ref[...]
        y = jnp.dot(y, w2_ref[...], preferred_element_type=jnp.float32) + c2_ref[...]
        out_ref[...] = y

    w_spec = pl.BlockSpec((H, H), lambda i: (0, 0))
    b_spec = pl.BlockSpec((1, H), lambda i: (0, 0))
    return pl.pallas_call(
        body,
        grid=(R // BR,),
        in_specs=[
            pl.BlockSpec((2, BR, H), lambda i: (0, i, 0)),
            pl.BlockSpec((2, BR), lambda i: (0, i)),
            pl.BlockSpec((BR, H), lambda i: (i, 0)),
            w_spec, w_spec, b_spec,
            w_spec, b_spec, w_spec, b_spec, w_spec, b_spec,
        ],
        out_specs=pl.BlockSpec((BR, H), lambda i: (i, 0)),
        out_shape=jax.ShapeDtypeStruct((R, H), jnp.float32),
    )(accp, degp, x, wl, wr, b.reshape(1, H), w0, c0.reshape(1, H),
      w1, c1.reshape(1, H), w2, c2.reshape(1, H))


def _sc_gather(y, idx):
    bpw = (2 * A) // NW
    mesh = plsc.VectorSubcoreMesh(**_MESH)

    @functools.partial(
        pl.kernel,
        out_type=jax.ShapeDtypeStruct((2 * A, H), jnp.float32),
        mesh=mesh,
        scratch_types=[
            pltpu.VMEM((1, bpw), jnp.int32),
            pltpu.VMEM((bpw, H), jnp.float32),
        ],
    )
    def kern(y_hbm, idx_hbm, out_hbm, idx_v, rows_v):
        cid = lax.axis_index("c")
        sid = lax.axis_index("s")
        base = (sid * 2 + cid) * bpw
        pltpu.sync_copy(idx_hbm.at[pl.ds(base, bpw)], idx_v.at[0])
        pltpu.sync_copy(y_hbm.at[idx_v.at[0]], rows_v)
        pltpu.sync_copy(rows_v, out_hbm.at[pl.ds(base, bpw)])

    return kern(y, idx)


def _tc_actions(g):
    def body(g_ref, out_ref):
        prod = g_ref[0:A, 0:NH] * g_ref[A:2 * A, NH:H]
        p = jnp.sum(prod, axis=0)
        m = jnp.max(p)
        e = jnp.exp(p - m)
        out_ref[0, :] = e / jnp.sum(e)

    return pl.pallas_call(
        body,
        out_shape=jax.ShapeDtypeStruct((1, NH), jnp.float32),
    )(g)


def kernel(actions, obs, eic, eid, eit, batch,
           W_l1, W_r1, b1, W_l2, W_r2, b2, W_l3, W_r3, b3,
           W_lin0, b_lin0, W_lin1, b_lin1, W_out, b_out):
    x = jnp.pad(obs, ((0, R - N), (0, 0)))
    z2 = jnp.zeros((R, H), jnp.float32)
    z1 = jnp.zeros((R,), jnp.float32)
    npad = EPAD - E
    gsrc = jnp.arange(npad, dtype=jnp.int32) % N
    gdst = N + (jnp.arange(npad, dtype=jnp.int32) % (R - N))

    def prep(e):
        s = jnp.concatenate([e[0], gsrc])
        d = jnp.concatenate([e[1], gdst])
        return s.reshape(ROWS_E, CH), d.reshape(ROWS_E, CH)

    s1, d1 = prep(eit)
    s2, d2 = prep(eic)
    s3, d3 = prep(eid)

    accp, degp = _sc_aggregate(x, s1, d1, z2, z1)
    x1 = _tc_conv(accp, degp, x, W_l1, W_r1, b1)
    accp, degp = _sc_aggregate(x1, s2, d2, z2, z1)
    x2 = _tc_conv(accp, degp, x1, W_l2, W_r2, b2)
    accp, degp = _sc_aggregate(x2, s3, d3, z2, z1)
    y = _tc_final(accp, degp, x2, W_l3, W_r3, b3,
                  W_lin0, b_lin0, W_lin1, b_lin1, W_out, b_out)

    idx = jnp.concatenate([actions[0, :, 0], actions[0, :, 1]])
    g = _sc_gather(y, idx.astype(jnp.int32))
    return _tc_actions(g)


# speedup vs baseline: 1.1652x; 1.0304x over previous
import functools

import jax
import jax.numpy as jnp
from jax import lax
from jax.experimental import pallas as pl
from jax.experimental.pallas import tpu as pltpu
from jax.experimental.pallas import tpu_sc as plsc

N = 10000
E = 320000
H = 128
NH = 64
A = 256
R = 10240
NW = 32
CH = 128
PH = 4
KP0 = 20
KP1 = 20
LW = 32
B1 = 16 * PH * KP0
ROWS_E = 16 * PH * (KP0 + KP1) + 8
EPAD = ROWS_E * CH
BR = 2048
STRIPE = R // 16

_MESH = dict(core_axis_name="c", subcore_axis_name="s", num_cores=2,
             num_subcores=16)


def _sc_aggregate(x, src, dst, z2, z1):
    mesh = plsc.VectorSubcoreMesh(**_MESH)

    @functools.partial(
        pl.kernel,
        out_type=(jax.ShapeDtypeStruct((2, R, H), jnp.float32),
                  jax.ShapeDtypeStruct((2, R), jnp.float32)),
        mesh=mesh,
        scratch_types=[
            pltpu.VMEM((LW, CH), jnp.int32),
            pltpu.VMEM((LW, CH), jnp.int32),
            pltpu.VMEM((2, CH, H), jnp.float32),
            pltpu.VMEM((CH,), jnp.float32),
            pltpu.VMEM_SHARED((R, H), jnp.float32),
            pltpu.VMEM_SHARED((R,), jnp.float32),
            pltpu.SemaphoreType.DMA,
            pltpu.SemaphoreType.DMA,
            pltpu.SemaphoreType.DMA,
        ],
    )
    def kern(x_hbm, src_hbm, dst_hbm, z2_hbm, z1_hbm, acc_out, deg_out,
             src_v, dst_v, rows_v, ones_v, acc_sh, deg_sh, gsem, ssem, dsem):
        cid = lax.axis_index("c")
        sid = lax.axis_index("s")
        pltpu.sync_copy(z2_hbm.at[pl.ds(sid * STRIPE, STRIPE)],
                        acc_sh.at[pl.ds(sid * STRIPE, STRIPE)])
        pltpu.sync_copy(z1_hbm.at[pl.ds(sid * STRIPE, STRIPE)],
                        deg_sh.at[pl.ds(sid * STRIPE, STRIPE)])
        for j in range(CH // 16):
            ones_v[pl.ds(j * 16, 16)] = jnp.ones((16,), jnp.float32)
        plsc.subcore_barrier()

        def gather(m):
            return pltpu.make_async_copy(
                x_hbm.at[src_v.at[m]], rows_v.at[m % 2], gsem)

        def acc_scatter(m):
            return pltpu.make_async_copy(
                rows_v.at[m % 2], acc_sh.at[dst_v.at[m]], ssem)

        def deg_scatter(m):
            return pltpu.make_async_copy(
                ones_v, deg_sh.at[dst_v.at[m]], dsem)

        def run(base, kp):
            for phase in range(PH):
                row0 = base + phase * kp
                start = pl.multiple_of((row0 // 8) * 8, 8)
                off = row0 - start
                pltpu.sync_copy(src_hbm.at[pl.ds(start, LW)], src_v)
                pltpu.sync_copy(dst_hbm.at[pl.ds(start, LW)], dst_v)
                gather(off).start()

                def body(m, carry):
                    gather(m + 1).start()
                    gather(m).wait()
                    deg_scatter(m).start(add=True)
                    acc_scatter(m).start(add=True)
                    acc_scatter(m).wait()
                    return carry

                lax.fori_loop(off, off + kp - 1, body, 0)
                gather(off + kp - 1).wait()
                deg_scatter(off + kp - 1).start(add=True)
                acc_scatter(off + kp - 1).start(add=True)
                acc_scatter(off + kp - 1).wait()

                def drain(m, carry):
                    deg_scatter(0).wait()
                    return carry

                lax.fori_loop(0, kp, drain, 0)

        @pl.when(cid == 0)
        def _():
            run(sid * (PH * KP0), KP0)

        @pl.when(cid == 1)
        def _():
            run(B1 + sid * (PH * KP1), KP1)

        plsc.subcore_barrier()
        pltpu.sync_copy(acc_sh.at[pl.ds(sid * STRIPE, STRIPE)],
                        acc_out.at[cid, pl.ds(sid * STRIPE, STRIPE)])
        pltpu.sync_copy(deg_sh.at[pl.ds(sid * STRIPE, STRIPE)],
                        deg_out.at[cid, pl.ds(sid * STRIPE, STRIPE)])

    return kern(x, src, dst, z2, z1)


def _conv_math(acc_ref, deg_ref, x_ref, wl_ref, wr_ref, b_ref):
    agg = acc_ref[0] + acc_ref[1]
    deg = deg_ref[0] + deg_ref[1]
    mean = agg / jnp.maximum(deg, 1.0)[:, None]
    return (jnp.dot(mean, wl_ref[...], preferred_element_type=jnp.float32)
            + jnp.dot(x_ref[...], wr_ref[...],
                      preferred_element_type=jnp.float32)
            + b_ref[...])


def _tc_conv(accp, degp, x, wl, wr, b):
    def body(acc_ref, deg_ref, x_ref, wl_ref, wr_ref, b_ref, out_ref):
        out_ref[...] = _conv_math(acc_ref, deg_ref, x_ref, wl_ref, wr_ref,
                                  b_ref)

    w_spec = pl.BlockSpec((H, H), lambda i: (0, 0))
    return pl.pallas_call(
        body,
        grid=(R // BR,),
        in_specs=[
            pl.BlockSpec((2, BR, H), lambda i: (0, i, 0)),
            pl.BlockSpec((2, BR), lambda i: (0, i)),
            pl.BlockSpec((BR, H), lambda i: (i, 0)),
            w_spec, w_spec,
            pl.BlockSpec((1, H), lambda i: (0, 0)),
        ],
        out_specs=pl.BlockSpec((BR, H), lambda i: (i, 0)),
        out_shape=jax.ShapeDtypeStruct((R, H), jnp.float32),
    )(accp, degp, x, wl, wr, b.reshape(1, H))


def _tc_final(accp, degp, x, wl, wr, b, w0, c0, w1, c1, w2, c2):
    def body(acc_ref, deg_ref, x_ref, wl_ref, wr_ref, b_ref,
             w0_ref, c0_ref, w1_ref, c1_ref, w2_ref, c2_ref, out_ref):
        y = _conv_math(acc_ref, deg_ref, x_ref, wl_ref, wr_ref, b_ref)
        y = jnp.dot(y, w0_ref[...], preferred_element_type=jnp.float32) + c0_ref[...]
        y = jnp.dot(y, w1_ref[...], preferred_element_type=jnp.float32) + c1_ref[...]
        y = jnp.dot(y, w2_ref[...], preferred_element_type=jnp.float32) + c2_ref[...]
        out_ref[...] = y

    w_spec = pl.BlockSpec((H, H), lambda i: (0, 0))
    b_spec = pl.BlockSpec((1, H), lambda i: (0, 0))
    return pl.pallas_call(
        body,
        grid=(R // BR,),
        in_specs=[
            pl.BlockSpec((2, BR, H), lambda i: (0, i, 0)),
            pl.BlockSpec((2, BR), lambda i: (0, i)),
            pl.BlockSpec((BR, H), lambda i: (i, 0)),
            w_spec, w_spec, b_spec,
            w_spec, b_spec, w_spec, b_spec, w_spec, b_spec,
        ],
        out_specs=pl.BlockSpec((BR, H), lambda i: (i, 0)),
        out_shape=jax.ShapeDtypeStruct((R, H), jnp.float32),
    )(accp, degp, x, wl, wr, b.reshape(1, H), w0, c0.reshape(1, H),
      w1, c1.reshape(1, H), w2, c2.reshape(1, H))


def _sc_gather(y, idx):
    bpw = (2 * A) // NW
    mesh = plsc.VectorSubcoreMesh(**_MESH)

    @functools.partial(
        pl.kernel,
        out_type=jax.ShapeDtypeStruct((2 * A, H), jnp.float32),
        mesh=mesh,
        scratch_types=[
            pltpu.VMEM((1, bpw), jnp.int32),
            pltpu.VMEM((bpw, H), jnp.float32),
        ],
    )
    def kern(y_hbm, idx_hbm, out_hbm, idx_v, rows_v):
        cid = lax.axis_index("c")
        sid = lax.axis_index("s")
        base = (sid * 2 + cid) * bpw
        pltpu.sync_copy(idx_hbm.at[pl.ds(base, bpw)], idx_v.at[0])
        pltpu.sync_copy(y_hbm.at[idx_v.at[0]], rows_v)
        pltpu.sync_copy(rows_v, out_hbm.at[pl.ds(base, bpw)])

    return kern(y, idx)


def _tc_actions(g):
    def body(g_ref, out_ref):
        prod = g_ref[0:A, 0:NH] * g_ref[A:2 * A, NH:H]
        p = jnp.sum(prod, axis=0)
        m = jnp.max(p)
        e = jnp.exp(p - m)
        out_ref[0, :] = e / jnp.sum(e)

    return pl.pallas_call(
        body,
        out_shape=jax.ShapeDtypeStruct((1, NH), jnp.float32),
    )(g)


def kernel(actions, obs, eic, eid, eit, batch,
           W_l1, W_r1, b1, W_l2, W_r2, b2, W_l3, W_r3, b3,
           W_lin0, b_lin0, W_lin1, b_lin1, W_out, b_out):
    x = jnp.pad(obs, ((0, R - N), (0, 0)))
    z2 = jnp.zeros((R, H), jnp.float32)
    z1 = jnp.zeros((R,), jnp.float32)
    npad = EPAD - E
    gsrc = jnp.arange(npad, dtype=jnp.int32) % N
    gdst = N + (jnp.arange(npad, dtype=jnp.int32) % (R - N))

    def prep(e):
        s = jnp.concatenate([e[0], gsrc])
        d = jnp.concatenate([e[1], gdst])
        return s.reshape(ROWS_E, CH), d.reshape(ROWS_E, CH)

    s1, d1 = prep(eit)
    s2, d2 = prep(eic)
    s3, d3 = prep(eid)

    accp, degp = _sc_aggregate(x, s1, d1, z2, z1)
    x1 = _tc_conv(accp, degp, x, W_l1, W_r1, b1)
    accp, degp = _sc_aggregate(x1, s2, d2, z2, z1)
    x2 = _tc_conv(accp, degp, x1, W_l2, W_r2, b2)
    accp, degp = _sc_aggregate(x2, s3, d3, z2, z1)
    y = _tc_final(accp, degp, x2, W_l3, W_r3, b3,
                  W_lin0, b_lin0, W_lin1, b_lin1, W_out, b_out)

    idx = jnp.concatenate([actions[0, :, 0], actions[0, :, 1]])
    g = _sc_gather(y, idx.astype(jnp.int32))
    return _tc_actions(g)


# speedup vs baseline: 1.2971x; 1.1132x over previous
import functools

import jax
import jax.numpy as jnp
from jax import lax
from jax.experimental import pallas as pl
from jax.experimental.pallas import tpu as pltpu
from jax.experimental.pallas import tpu_sc as plsc

N = 10000
E = 320000
H = 128
NH = 64
A = 256
R = 10240
NW = 32
CH = 128
PH = 4
LW = 32
ROWS_E = E // CH
AUX0 = 2480
BR = 2048
STRIPE = R // 16

_MESH = dict(core_axis_name="c", subcore_axis_name="s", num_cores=2,
             num_subcores=16)


def _sc_aggregate(x, e, aux, z2, z1):
    mesh = plsc.VectorSubcoreMesh(**_MESH)

    @functools.partial(
        pl.kernel,
        out_type=(jax.ShapeDtypeStruct((2, R, H), jnp.float32),
                  jax.ShapeDtypeStruct((2, R), jnp.float32)),
        mesh=mesh,
        scratch_types=[
            pltpu.VMEM((2, LW * CH), jnp.int32),
            pltpu.VMEM((2, CH, H), jnp.float32),
            pltpu.VMEM((CH,), jnp.float32),
            pltpu.VMEM_SHARED((R, H), jnp.float32),
            pltpu.VMEM_SHARED((R,), jnp.float32),
            pltpu.SemaphoreType.DMA,
            pltpu.SemaphoreType.DMA,
            pltpu.SemaphoreType.DMA,
        ],
    )
    def kern(x_hbm, e_hbm, aux_hbm, z2_hbm, z1_hbm, acc_out, deg_out,
             ev, rows_v, ones_v, acc_sh, deg_sh, gsem, ssem, dsem):
        cid = lax.axis_index("c")
        sid = lax.axis_index("s")
        pltpu.sync_copy(z2_hbm.at[pl.ds(sid * STRIPE, STRIPE)],
                        acc_sh.at[pl.ds(sid * STRIPE, STRIPE)])
        pltpu.sync_copy(z1_hbm.at[pl.ds(sid * STRIPE, STRIPE)],
                        deg_sh.at[pl.ds(sid * STRIPE, STRIPE)])
        for j in range(CH // 16):
            ones_v[pl.ds(j * 16, 16)] = jnp.ones((16,), jnp.float32)
        plsc.subcore_barrier()

        def gather(m):
            return pltpu.make_async_copy(
                x_hbm.at[ev.at[0, pl.ds(m * CH, CH)]], rows_v.at[m % 2],
                gsem)

        def acc_scatter(m):
            return pltpu.make_async_copy(
                rows_v.at[m % 2], acc_sh.at[ev.at[1, pl.ds(m * CH, CH)]],
                ssem)

        def deg_scatter(m):
            return pltpu.make_async_copy(
                ones_v, deg_sh.at[ev.at[1, pl.ds(m * CH, CH)]], dsem)

        def run(base, counts, aux_last):
            for phase, kp in enumerate(counts):
                row0 = base + sum(counts[:phase])
                if aux_last and phase == len(counts) - 1:
                    pltpu.sync_copy(aux_hbm, ev)
                    off = row0 - AUX0
                else:
                    start = pl.multiple_of((row0 // 8) * 8, 8)
                    off = row0 - start
                    pltpu.sync_copy(
                        e_hbm.at[:, pl.ds(start * CH, LW * CH)], ev)
                gather(off).start()

                def body(m, carry):
                    gather(m + 1).start()
                    gather(m).wait()
                    deg_scatter(m).start(add=True)
                    acc_scatter(m).start(add=True)
                    acc_scatter(m).wait()
                    return carry

                lax.fori_loop(off, off + kp - 1, body, 0)
                gather(off + kp - 1).wait()
                deg_scatter(off + kp - 1).start(add=True)
                acc_scatter(off + kp - 1).start(add=True)
                acc_scatter(off + kp - 1).wait()

                def drain(m, carry):
                    deg_scatter(0).wait()
                    return carry

                lax.fori_loop(0, kp, drain, 0)

        @pl.when((cid == 0) & (sid < 4))
        def _():
            run(79 * sid, (20, 20, 20, 19), False)

        @pl.when((cid == 0) & (sid >= 4))
        def _():
            run(78 * sid + 4, (20, 20, 19, 19), False)

        @pl.when((cid == 1) & (sid < 15))
        def _():
            run(1252 + 78 * sid, (20, 20, 19, 19), False)

        @pl.when((cid == 1) & (sid == 15))
        def _():
            run(1252 + 78 * 15, (20, 20, 19, 19), True)

        plsc.subcore_barrier()
        pltpu.sync_copy(acc_sh.at[pl.ds(sid * STRIPE, STRIPE)],
                        acc_out.at[cid, pl.ds(sid * STRIPE, STRIPE)])
        pltpu.sync_copy(deg_sh.at[pl.ds(sid * STRIPE, STRIPE)],
                        deg_out.at[cid, pl.ds(sid * STRIPE, STRIPE)])

    return kern(x, e, aux, z2, z1)


def _conv_math(acc_ref, deg_ref, x_ref, wl_ref, wr_ref, b_ref):
    agg = acc_ref[0] + acc_ref[1]
    deg = deg_ref[0] + deg_ref[1]
    mean = agg / jnp.maximum(deg, 1.0)[:, None]
    return (jnp.dot(mean, wl_ref[...], preferred_element_type=jnp.float32)
            + jnp.dot(x_ref[...], wr_ref[...],
                      preferred_element_type=jnp.float32)
            + b_ref[...])


def _tc_conv(accp, degp, x, wl, wr, b):
    def body(acc_ref, deg_ref, x_ref, wl_ref, wr_ref, b_ref, out_ref):
        out_ref[...] = _conv_math(acc_ref, deg_ref, x_ref, wl_ref, wr_ref,
                                  b_ref)

    w_spec = pl.BlockSpec((H, H), lambda i: (0, 0))
    return pl.pallas_call(
        body,
        grid=(R // BR,),
        in_specs=[
            pl.BlockSpec((2, BR, H), lambda i: (0, i, 0)),
            pl.BlockSpec((2, BR), lambda i: (0, i)),
            pl.BlockSpec((BR, H), lambda i: (i, 0)),
            w_spec, w_spec,
            pl.BlockSpec((1, H), lambda i: (0, 0)),
        ],
        out_specs=pl.BlockSpec((BR, H), lambda i: (i, 0)),
        out_shape=jax.ShapeDtypeStruct((R, H), jnp.float32),
    )(accp, degp, x, wl, wr, b.reshape(1, H))


def _tc_final(accp, degp, x, wl, wr, b, w0, c0, w1, c1, w2, c2):
    def body(acc_ref, deg_ref, x_ref, wl_ref, wr_ref, b_ref,
             w0_ref, c0_ref, w1_ref, c1_ref, w2_ref, c2_ref, out_ref):
        y = _conv_math(acc_ref, deg_ref, x_ref, wl_ref, wr_ref, b_ref)
        y = jnp.dot(y, w0_ref[...], preferred_element_type=jnp.float32) + c0_ref[...]
        y = jnp.dot(y, w1_ref[...], preferred_element_type=jnp.float32) + c1_ref[...]
        y = jnp.dot(y, w2_ref[...], preferred_element_type=jnp.float32) + c2_ref[...]
        out_ref[...] = y

    w_spec = pl.BlockSpec((H, H), lambda i: (0, 0))
    b_spec = pl.BlockSpec((1, H), lambda i: (0, 0))
    return pl.pallas_call(
        body,
        grid=(R // BR,),
        in_specs=[
            pl.BlockSpec((2, BR, H), lambda i: (0, i, 0)),
            pl.BlockSpec((2, BR), lambda i: (0, i)),
            pl.BlockSpec((BR, H), lambda i: (i, 0)),
            w_spec, w_spec, b_spec,
            w_spec, b_spec, w_spec, b_spec, w_spec, b_spec,
        ],
        out_specs=pl.BlockSpec((BR, H), lambda i: (i, 0)),
        out_shape=jax.ShapeDtypeStruct((R, H), jnp.float32),
    )(accp, degp, x, wl, wr, b.reshape(1, H), w0, c0.reshape(1, H),
      w1, c1.reshape(1, H), w2, c2.reshape(1, H))


def _sc_gather(y, idx):
    bpw = (2 * A) // NW
    mesh = plsc.VectorSubcoreMesh(**_MESH)

    @functools.partial(
        pl.kernel,
        out_type=jax.ShapeDtypeStruct((2 * A, H), jnp.float32),
        mesh=mesh,
        scratch_types=[
            pltpu.VMEM((1, bpw), jnp.int32),
            pltpu.VMEM((bpw, H), jnp.float32),
        ],
    )
    def kern(y_hbm, idx_hbm, out_hbm, idx_v, rows_v):
        cid = lax.axis_index("c")
        sid = lax.axis_index("s")
        base = (sid * 2 + cid) * bpw
        pltpu.sync_copy(idx_hbm.at[pl.ds(base, bpw)], idx_v.at[0])
        pltpu.sync_copy(y_hbm.at[idx_v.at[0]], rows_v)
        pltpu.sync_copy(rows_v, out_hbm.at[pl.ds(base, bpw)])

    return kern(y, idx)


def _tc_actions(g):
    def body(g_ref, out_ref):
        prod = g_ref[0:A, 0:NH] * g_ref[A:2 * A, NH:H]
        p = jnp.sum(prod, axis=0)
        m = jnp.max(p)
        e = jnp.exp(p - m)
        out_ref[0, :] = e / jnp.sum(e)

    return pl.pallas_call(
        body,
        out_shape=jax.ShapeDtypeStruct((1, NH), jnp.float32),
    )(g)


def kernel(actions, obs, eic, eid, eit, batch,
           W_l1, W_r1, b1, W_l2, W_r2, b2, W_l3, W_r3, b3,
           W_lin0, b_lin0, W_lin1, b_lin1, W_out, b_out):
    x = jnp.pad(obs, ((0, R - N), (0, 0)))
    z2 = jnp.zeros((R, H), jnp.float32)
    z1 = jnp.zeros((R,), jnp.float32)

    def tail(e):
        return jnp.pad(e[:, AUX0 * CH:], ((0, 0), (0, (AUX0 + LW) * CH - E)))

    accp, degp = _sc_aggregate(x, eit, tail(eit), z2, z1)
    x1 = _tc_conv(accp, degp, x, W_l1, W_r1, b1)
    accp, degp = _sc_aggregate(x1, eic, tail(eic), z2, z1)
    x2 = _tc_conv(accp, degp, x1, W_l2, W_r2, b2)
    accp, degp = _sc_aggregate(x2, eid, tail(eid), z2, z1)
    y = _tc_final(accp, degp, x2, W_l3, W_r3, b3,
                  W_lin0, b_lin0, W_lin1, b_lin1, W_out, b_out)

    idx = jnp.concatenate([actions[0, :, 0], actions[0, :, 1]])
    g = _sc_gather(y, idx.astype(jnp.int32))
    return _tc_actions(g)
